# baseline (speedup 1.0000x reference)
import functools

import jax
import jax.numpy as jnp
from jax import lax
from jax.experimental import pallas as pl
from jax.experimental.pallas import tpu as pltpu
from jax.experimental.pallas import tpu_sc as plsc

H = 128
NN = 100000

NW = 32
CH = 128
NCH0 = 45
NCH1 = 5
MAXC = max(NCH0, NCH1)
TOT = 16 * (NCH0 + NCH1) * CH
IDXPAD = TOT + MAXC * CH

NB = 4


def _sc_gather_body(hc_hbm, i0_hbm, i1_hbm, o0, o1,
                    i0_v, i1_v, b0, b1, b2, b3, s0, s1, s2, s3):
    bufs = (b0, b1, b2, b3)
    sems = (s0, s1, s2, s3)
    cc = lax.axis_index("c")
    ss = lax.axis_index("s")
    nch = jnp.where(cc == 0, NCH0, NCH1)
    base = (jnp.where(cc == 0, ss * NCH0, 16 * NCH0 + ss * NCH1) * CH)
    ngrp = (nch + NB - 1) // NB
    pltpu.sync_copy(i0_hbm.at[pl.ds(base, MAXC * CH)], i0_v)
    pltpu.sync_copy(i1_hbm.at[pl.ds(base, MAXC * CH)], i1_v)
    for idx_v, out in ((i0_v, o0), (i1_v, o1)):
        def gstart(j, b, idx_v=idx_v):
            pltpu.async_copy(
                hc_hbm.at[idx_v.at[pl.ds(j * CH, CH)]], bufs[b], sems[b])

        for b in range(NB):
            @pl.when(b < nch)
            def _(b=b, gstart=gstart):
                gstart(b, b)

        def body(i, _, idx_v=idx_v, out=out, gstart=gstart):
            for b in range(NB):
                j = i * NB + b

                @pl.when(j < nch)
                def _(j=j, b=b):
                    pltpu.make_async_copy(
                        hc_hbm.at[idx_v.at[pl.ds(j * CH, CH)]], bufs[b],
                        sems[b]).wait()
                    pltpu.sync_copy(bufs[b],
                                    out.at[pl.ds(base + j * CH, CH)])

                @pl.when(j + NB < nch)
                def _(j=j, b=b):
                    gstart(j + NB, b)
            return 0
        lax.fori_loop(0, ngrp, body, 0)


@functools.cache
def _sc_gather_kernel():
    mesh = plsc.VectorSubcoreMesh(core_axis_name="c", subcore_axis_name="s")
    return pl.kernel(
        _sc_gather_body,
        mesh=mesh,
        out_type=[jax.ShapeDtypeStruct((TOT, H), jnp.float32)] * 2,
        scratch_types=(
            [pltpu.VMEM((MAXC * CH,), jnp.int32)] * 2
            + [pltpu.VMEM((CH, H), jnp.float32)] * NB
            + [pltpu.SemaphoreType.DMA] * NB
        ),
    )


BN = 2000
GRID = NN // BN


def _unpack(ref):
    w = lax.bitcast_convert_type(ref[...], jnp.uint32)
    hv = lax.bitcast_convert_type(w & jnp.uint32(0xFFFF0000), jnp.float32)
    cv = lax.bitcast_convert_type(w << 16, jnp.float32)
    return hv, cv


def _tc_body(g0_ref, g1_ref, iou_ref, w_ref, ufb_ref,
             biou_ref, siou_ref, sc_ref, ho_ref, co_ref):
    h0, c0 = _unpack(g0_ref)
    h1, c1 = _unpack(g1_ref)
    ia, ib = _unpack(iou_ref)
    w = w_ref[...]
    y = (jnp.dot(h0, w[:H, :], preferred_element_type=jnp.float32)
         + jnp.dot(h1, w[H:, :], preferred_element_type=jnp.float32))
    ufb = ufb_ref[...]
    f0 = jax.nn.sigmoid(y[:, :H] + ufb[:, :H])
    f1 = jax.nn.sigmoid(y[:, H:2 * H] + ufb[:, H:])
    c_red = f0 * c0 + f1 * c1
    hnorm = jnp.sqrt(jnp.sum(h0 * h0 + h1 * h1, axis=1, keepdims=True))
    iounorm = jnp.sqrt(jnp.sum(ia * ia + ib * ib, axis=1, keepdims=True))
    s = iounorm * siou_ref[0, 0] / jnp.maximum(hnorm, 1e-12)
    iou_b = y[:, 2 * H:] * s + biou_ref[...]
    c0norm = jnp.sqrt(jnp.sum(c0 * c0, axis=1, keepdims=True))
    crnorm = jnp.maximum(
        jnp.sqrt(jnp.sum(c_red * c_red, axis=1, keepdims=True)), 1e-12)
    c_data = c_red * (c0norm * sc_ref[0, 0] / crnorm)
    gi = jax.nn.sigmoid(iou_b[:, :H])
    go = jax.nn.sigmoid(iou_b[:, H:2 * H])
    gu = jnp.tanh(iou_b[:, 2 * H:])
    c_out = gi * gu + c_data
    co_ref[...] = c_out
    ho_ref[...] = go * jnp.tanh(c_out)


def _tc_call(g0, g1, iou, wcat, ufb, biou, siou, sc):
    row = lambda i: (i, 0)
    zero = lambda i: (0, 0)
    return pl.pallas_call(
        _tc_body,
        grid=(GRID,),
        in_specs=[
            pl.BlockSpec((BN, H), row),
            pl.BlockSpec((BN, H), row),
            pl.BlockSpec((BN, 3 * H // 2), row),
            pl.BlockSpec((2 * H, 5 * H), zero),
            pl.BlockSpec((1, 2 * H), zero),
            pl.BlockSpec((1, 3 * H), zero),
            pl.BlockSpec((1, 1), zero, memory_space=pltpu.SMEM),
            pl.BlockSpec((1, 1), zero, memory_space=pltpu.SMEM),
        ],
        out_specs=[pl.BlockSpec((BN, H), row), pl.BlockSpec((BN, H), row)],
        out_shape=[jax.ShapeDtypeStruct((NN, H), jnp.float32)] * 2,
    )(g0, g1, iou, wcat, ufb, biou, siou, sc)


def kernel(h, c, iou, children, U_iou_w, b_iou, U_f_w, U_f_b, scale_iou,
           scale_c):
    hbits = lax.bitcast_convert_type(h, jnp.uint32)
    cbits = lax.bitcast_convert_type(c, jnp.uint32)
    hr = (hbits + jnp.uint32(0x8000)) & jnp.uint32(0xFFFF0000)
    cr = (cbits + jnp.uint32(0x8000)) >> 16
    hc = lax.bitcast_convert_type(hr | cr, jnp.float32)
    ibits = lax.bitcast_convert_type(iou, jnp.uint32)
    ir = (ibits[:, 0::2] + jnp.uint32(0x8000)) & jnp.uint32(0xFFFF0000)
    il = (ibits[:, 1::2] + jnp.uint32(0x8000)) >> 16
    iou_p = lax.bitcast_convert_type(ir | il, jnp.float32)
    ch = children.astype(jnp.int32)
    pad = jnp.zeros((IDXPAD - NN,), jnp.int32)
    i0 = jnp.concatenate([ch[:, 0], pad])
    i1 = jnp.concatenate([ch[:, 1], pad])
    g0, g1 = _sc_gather_kernel()(hc, i0, i1)
    wcat = jnp.concatenate([U_f_w.T, U_iou_w.T], axis=1)
    ufb = U_f_b.reshape(1, 2 * H)
    biou = b_iou.reshape(1, 3 * H)
    siou = scale_iou.reshape(1, 1)
    sc = scale_c.reshape(1, 1)
    h_out, c_out = _tc_call(g0, g1, iou_p, wcat, ufb, biou, siou, sc)
    return h_out, c_out


# speedup vs baseline: 7.8783x; 7.8783x over previous
import functools

import jax
import jax.numpy as jnp
from jax import lax
from jax.experimental import pallas as pl
from jax.experimental.pallas import tpu as pltpu
from jax.experimental.pallas import tpu_sc as plsc

H = 128
NN = 100000

NW = 32
CH = 128
NCH0 = 45
NCH1 = 5
MAXC = max(NCH0, NCH1)
TOT = 16 * (NCH0 + NCH1) * CH
IDXPAD = TOT + MAXC * CH

NB = 4


def _sc_gather_body(hc_hbm, i0_hbm, i1_hbm, o0, o1,
                    i0_v, i1_v, b0, b1, b2, b3, s0, s1, s2, s3):
    bufs = (b0, b1, b2, b3)
    sems = (s0, s1, s2, s3)
    cc = lax.axis_index("c")
    ss = lax.axis_index("s")
    nch = jnp.where(cc == 0, NCH0, NCH1)
    base = (jnp.where(cc == 0, ss * NCH0, 16 * NCH0 + ss * NCH1) * CH)
    ngrp = (nch + NB - 1) // NB
    pltpu.sync_copy(i0_hbm.at[pl.ds(base, MAXC * CH)], i0_v)
    pltpu.sync_copy(i1_hbm.at[pl.ds(base, MAXC * CH)], i1_v)
    for idx_v, out in ((i0_v, o0), (i1_v, o1)):
        def gstart(j, b, idx_v=idx_v):
            pltpu.async_copy(
                hc_hbm.at[idx_v.at[pl.ds(j * CH, CH)]], bufs[b], sems[b])

        for b in range(NB):
            @pl.when(b < nch)
            def _(b=b, gstart=gstart):
                gstart(b, b)

        def body(i, _, idx_v=idx_v, out=out, gstart=gstart):
            for b in range(NB):
                j = i * NB + b

                @pl.when(j < nch)
                def _(j=j, b=b):
                    pltpu.make_async_copy(
                        hc_hbm.at[idx_v.at[pl.ds(j * CH, CH)]], bufs[b],
                        sems[b]).wait()
                    pltpu.sync_copy(bufs[b],
                                    out.at[pl.ds(base + j * CH, CH)])

                @pl.when(j + NB < nch)
                def _(j=j, b=b):
                    gstart(j + NB, b)
            return 0
        lax.fori_loop(0, ngrp, body, 0)


@functools.cache
def _sc_gather_kernel():
    mesh = plsc.VectorSubcoreMesh(core_axis_name="c", subcore_axis_name="s")
    return pl.kernel(
        _sc_gather_body,
        mesh=mesh,
        out_type=[jax.ShapeDtypeStruct((TOT, H), jnp.float32)] * 2,
        scratch_types=(
            [pltpu.VMEM((MAXC * CH,), jnp.int32)] * 2
            + [pltpu.VMEM((CH, H), jnp.float32)] * NB
            + [pltpu.SemaphoreType.DMA] * NB
        ),
    )


BN = 2000
GRID = NN // BN


def _unpack(ref):
    w = lax.bitcast_convert_type(ref[...], jnp.uint32)
    hv = lax.bitcast_convert_type(w & jnp.uint32(0xFFFF0000), jnp.float32)
    cv = lax.bitcast_convert_type(w << 16, jnp.float32)
    return hv, cv


def _tc_body(g0_ref, g1_ref, iou_ref, w_ref, ufb_ref,
             biou_ref, siou_ref, sc_ref, ho_ref, co_ref):
    h0, c0 = _unpack(g0_ref)
    h1, c1 = _unpack(g1_ref)
    ia, ib = _unpack(iou_ref)
    w = w_ref[...]
    y = (jnp.dot(h0, w[:H, :], preferred_element_type=jnp.float32)
         + jnp.dot(h1, w[H:, :], preferred_element_type=jnp.float32))
    ufb = ufb_ref[...]
    f0 = jax.nn.sigmoid(y[:, :H] + ufb[:, :H])
    f1 = jax.nn.sigmoid(y[:, H:2 * H] + ufb[:, H:])
    c_red = f0 * c0 + f1 * c1
    hnorm = jnp.sqrt(jnp.sum(h0 * h0 + h1 * h1, axis=1, keepdims=True))
    iounorm = jnp.sqrt(jnp.sum(ia * ia + ib * ib, axis=1, keepdims=True))
    s = iounorm * siou_ref[0, 0] / jnp.maximum(hnorm, 1e-12)
    iou_b = y[:, 2 * H:] * s + biou_ref[...]
    c0norm = jnp.sqrt(jnp.sum(c0 * c0, axis=1, keepdims=True))
    crnorm = jnp.maximum(
        jnp.sqrt(jnp.sum(c_red * c_red, axis=1, keepdims=True)), 1e-12)
    c_data = c_red * (c0norm * sc_ref[0, 0] / crnorm)
    gi = jax.nn.sigmoid(iou_b[:, :H])
    go = jax.nn.sigmoid(iou_b[:, H:2 * H])
    gu = jnp.tanh(iou_b[:, 2 * H:])
    c_out = gi * gu + c_data
    co_ref[...] = c_out
    ho_ref[...] = go * jnp.tanh(c_out)


def _tc_call(g0, g1, iou, wcat, ufb, biou, siou, sc):
    row = lambda i: (i, 0)
    zero = lambda i: (0, 0)
    return pl.pallas_call(
        _tc_body,
        grid=(GRID,),
        in_specs=[
            pl.BlockSpec((BN, H), row),
            pl.BlockSpec((BN, H), row),
            pl.BlockSpec((BN, 3 * H // 2), row),
            pl.BlockSpec((2 * H, 5 * H), zero),
            pl.BlockSpec((1, 2 * H), zero),
            pl.BlockSpec((1, 3 * H), zero),
            pl.BlockSpec((1, 1), zero, memory_space=pltpu.SMEM),
            pl.BlockSpec((1, 1), zero, memory_space=pltpu.SMEM),
        ],
        out_specs=[pl.BlockSpec((BN, H), row), pl.BlockSpec((BN, H), row)],
        out_shape=[jax.ShapeDtypeStruct((NN, H), jnp.float32)] * 2,
    )(g0, g1, iou, wcat, ufb, biou, siou, sc)


def kernel(h, c, iou, children, U_iou_w, b_iou, U_f_w, U_f_b, scale_iou,
           scale_c):
    hbits = lax.bitcast_convert_type(h, jnp.uint32)
    cbits = lax.bitcast_convert_type(c, jnp.uint32)
    hr = (hbits + jnp.uint32(0x8000)) & jnp.uint32(0xFFFF0000)
    cr = (cbits + jnp.uint32(0x8000)) >> 16
    hc = lax.bitcast_convert_type(hr | cr, jnp.float32)
    ibits = lax.bitcast_convert_type(iou, jnp.uint32)
    ir = (ibits[:, :3 * H // 2] + jnp.uint32(0x8000)) & jnp.uint32(0xFFFF0000)
    il = (ibits[:, 3 * H // 2:] + jnp.uint32(0x8000)) >> 16
    iou_p = lax.bitcast_convert_type(ir | il, jnp.float32)
    ch = children.astype(jnp.int32)
    pad = jnp.zeros((IDXPAD - NN,), jnp.int32)
    i0 = jnp.concatenate([ch[:, 0], pad])
    i1 = jnp.concatenate([ch[:, 1], pad])
    g0, g1 = _sc_gather_kernel()(hc, i0, i1)
    wcat = jnp.concatenate([U_f_w.T, U_iou_w.T], axis=1)
    ufb = U_f_b.reshape(1, 2 * H)
    biou = b_iou.reshape(1, 3 * H)
    siou = scale_iou.reshape(1, 1)
    sc = scale_c.reshape(1, 1)
    h_out, c_out = _tc_call(g0, g1, iou_p, wcat, ufb, biou, siou, sc)
    return h_out, c_out


# speedup vs baseline: 10.0109x; 1.2707x over previous
import functools

import jax
import jax.numpy as jnp
from jax import lax
from jax.experimental import pallas as pl
from jax.experimental.pallas import tpu as pltpu
from jax.experimental.pallas import tpu_sc as plsc

H = 128
NN = 100000

NW = 32
CH = 128
NCH0 = 45
NCH1 = 5
MAXC = max(NCH0, NCH1)
TOT = 16 * (NCH0 + NCH1) * CH
IDXPAD = TOT + MAXC * CH

NB = 4


def _sc_gather_body(hc_hbm, i0_hbm, i1_hbm, o0, o1,
                    i0_v, i1_v, b0, b1, b2, b3, s0, s1, s2, s3):
    bufs = (b0, b1, b2, b3)
    sems = (s0, s1, s2, s3)
    cc = lax.axis_index("c")
    ss = lax.axis_index("s")
    nch = jnp.where(cc == 0, NCH0, NCH1)
    base = (jnp.where(cc == 0, ss * NCH0, 16 * NCH0 + ss * NCH1) * CH)
    ngrp = (nch + NB - 1) // NB
    pltpu.sync_copy(i0_hbm.at[pl.ds(base, MAXC * CH)], i0_v)
    pltpu.sync_copy(i1_hbm.at[pl.ds(base, MAXC * CH)], i1_v)
    for idx_v, out in ((i0_v, o0), (i1_v, o1)):
        def gstart(j, b, idx_v=idx_v):
            pltpu.async_copy(
                hc_hbm.at[idx_v.at[pl.ds(j * CH, CH)]], bufs[b], sems[b])

        for b in range(NB):
            @pl.when(b < nch)
            def _(b=b, gstart=gstart):
                gstart(b, b)

        def body(i, _, idx_v=idx_v, out=out, gstart=gstart):
            for b in range(NB):
                j = i * NB + b

                @pl.when(j < nch)
                def _(j=j, b=b):
                    pltpu.make_async_copy(
                        hc_hbm.at[idx_v.at[pl.ds(j * CH, CH)]], bufs[b],
                        sems[b]).wait()
                    pltpu.sync_copy(bufs[b],
                                    out.at[pl.ds(base + j * CH, CH)])

                @pl.when(j + NB < nch)
                def _(j=j, b=b):
                    gstart(j + NB, b)
            return 0
        lax.fori_loop(0, ngrp, body, 0)


@functools.cache
def _sc_gather_kernel():
    mesh = plsc.VectorSubcoreMesh(core_axis_name="c", subcore_axis_name="s")
    return pl.kernel(
        _sc_gather_body,
        mesh=mesh,
        out_type=[jax.ShapeDtypeStruct((TOT, H), jnp.float32)] * 2,
        scratch_types=(
            [pltpu.VMEM((MAXC * CH,), jnp.int32)] * 2
            + [pltpu.VMEM((CH, H), jnp.float32)] * NB
            + [pltpu.SemaphoreType.DMA] * NB
        ),
    )


BN = 2000
GRID = NN // BN


def _unpack(ref):
    w = lax.bitcast_convert_type(ref[...], jnp.uint32)
    hv = lax.bitcast_convert_type(w & jnp.uint32(0xFFFF0000), jnp.float32)
    cv = lax.bitcast_convert_type(w << 16, jnp.float32)
    return hv, cv


def _norm_body(iou_ref, n2_ref):
    x = iou_ref[...]
    ones = jnp.ones((1, 3 * H), jnp.float32)
    n2 = lax.dot_general(ones, x * x, (((1,), (1,)), ((), ())),
                         preferred_element_type=jnp.float32)
    n2_ref[...] = n2.reshape(1, 1, BN)


def _norm_call(iou):
    return pl.pallas_call(
        _norm_body,
        grid=(GRID,),
        in_specs=[pl.BlockSpec((BN, 3 * H), lambda i: (i, 0))],
        out_specs=pl.BlockSpec((1, 1, BN), lambda i: (i, 0, 0)),
        out_shape=jax.ShapeDtypeStruct((GRID, 1, BN), jnp.float32),
    )(iou)


def _tc_body(g0_ref, g1_ref, n2_ref, w_ref, ufb_ref,
             biou_ref, siou_ref, sc_ref, ho_ref, co_ref):
    h0, c0 = _unpack(g0_ref)
    h1, c1 = _unpack(g1_ref)
    w = w_ref[...]
    y = (jnp.dot(h0, w[:H, :], preferred_element_type=jnp.float32)
         + jnp.dot(h1, w[H:, :], preferred_element_type=jnp.float32))
    ufb = ufb_ref[...]
    f0 = jax.nn.sigmoid(y[:, :H] + ufb[:, :H])
    f1 = jax.nn.sigmoid(y[:, H:2 * H] + ufb[:, H:])
    c_red = f0 * c0 + f1 * c1
    hnorm = jnp.sqrt(jnp.sum(h0 * h0 + h1 * h1, axis=1, keepdims=True))
    iounorm = jnp.sqrt(jnp.reshape(n2_ref[...], (BN, 1)))
    s = iounorm * siou_ref[0, 0] / jnp.maximum(hnorm, 1e-12)
    iou_b = y[:, 2 * H:] * s + biou_ref[...]
    c0norm = jnp.sqrt(jnp.sum(c0 * c0, axis=1, keepdims=True))
    crnorm = jnp.maximum(
        jnp.sqrt(jnp.sum(c_red * c_red, axis=1, keepdims=True)), 1e-12)
    c_data = c_red * (c0norm * sc_ref[0, 0] / crnorm)
    gi = jax.nn.sigmoid(iou_b[:, :H])
    go = jax.nn.sigmoid(iou_b[:, H:2 * H])
    gu = jnp.tanh(iou_b[:, 2 * H:])
    c_out = gi * gu + c_data
    co_ref[...] = c_out
    ho_ref[...] = go * jnp.tanh(c_out)


def _tc_call(g0, g1, n2, wcat, ufb, biou, siou, sc):
    row = lambda i: (i, 0)
    zero = lambda i: (0, 0)
    return pl.pallas_call(
        _tc_body,
        grid=(GRID,),
        in_specs=[
            pl.BlockSpec((BN, H), row),
            pl.BlockSpec((BN, H), row),
            pl.BlockSpec((1, 1, BN), lambda i: (i, 0, 0)),
            pl.BlockSpec((2 * H, 5 * H), zero),
            pl.BlockSpec((1, 2 * H), zero),
            pl.BlockSpec((1, 3 * H), zero),
            pl.BlockSpec((1, 1), zero, memory_space=pltpu.SMEM),
            pl.BlockSpec((1, 1), zero, memory_space=pltpu.SMEM),
        ],
        out_specs=[pl.BlockSpec((BN, H), row), pl.BlockSpec((BN, H), row)],
        out_shape=[jax.ShapeDtypeStruct((NN, H), jnp.float32)] * 2,
    )(g0, g1, n2, wcat, ufb, biou, siou, sc)


def kernel(h, c, iou, children, U_iou_w, b_iou, U_f_w, U_f_b, scale_iou,
           scale_c):
    hbits = lax.bitcast_convert_type(h, jnp.uint32)
    cbits = lax.bitcast_convert_type(c, jnp.uint32)
    hr = (hbits + jnp.uint32(0x8000)) & jnp.uint32(0xFFFF0000)
    cr = (cbits + jnp.uint32(0x8000)) >> 16
    hc = lax.bitcast_convert_type(hr | cr, jnp.float32)
    ch = children.astype(jnp.int32)
    pad = jnp.zeros((IDXPAD - NN,), jnp.int32)
    i0 = jnp.concatenate([ch[:, 0], pad])
    i1 = jnp.concatenate([ch[:, 1], pad])
    g0, g1 = _sc_gather_kernel()(hc, i0, i1)
    n2 = _norm_call(iou)
    wcat = jnp.concatenate([U_f_w.T, U_iou_w.T], axis=1)
    ufb = U_f_b.reshape(1, 2 * H)
    biou = b_iou.reshape(1, 3 * H)
    siou = scale_iou.reshape(1, 1)
    sc = scale_c.reshape(1, 1)
    h_out, c_out = _tc_call(g0, g1, n2, wcat, ufb, biou, siou, sc)
    return h_out, c_out


# speedup vs baseline: 10.0159x; 1.0005x over previous
import functools

import jax
import jax.numpy as jnp
from jax import lax
from jax.experimental import pallas as pl
from jax.experimental.pallas import tpu as pltpu
from jax.experimental.pallas import tpu_sc as plsc

H = 128
NN = 100000

NW = 32
CH = 128
NCH0 = 45
NCH1 = 5
MAXC = max(NCH0, NCH1)
TOT = 16 * (NCH0 + NCH1) * CH
IDXPAD = TOT + MAXC * CH

NB = 4


def _sc_gather_body(hc_hbm, i0_hbm, i1_hbm, o0, o1,
                    i0_v, i1_v, b0, b1, b2, b3, s0, s1, s2, s3):
    bufs = (b0, b1, b2, b3)
    sems = (s0, s1, s2, s3)
    cc = lax.axis_index("c")
    ss = lax.axis_index("s")
    nch = jnp.where(cc == 0, NCH0, NCH1)
    base = (jnp.where(cc == 0, ss * NCH0, 16 * NCH0 + ss * NCH1) * CH)
    ngrp = (nch + NB - 1) // NB
    pltpu.sync_copy(i0_hbm.at[pl.ds(base, MAXC * CH)], i0_v)
    pltpu.sync_copy(i1_hbm.at[pl.ds(base, MAXC * CH)], i1_v)
    for idx_v, out in ((i0_v, o0), (i1_v, o1)):
        def gstart(j, b, idx_v=idx_v):
            pltpu.async_copy(
                hc_hbm.at[idx_v.at[pl.ds(j * CH, CH)]], bufs[b], sems[b])

        for b in range(NB):
            @pl.when(b < nch)
            def _(b=b, gstart=gstart):
                gstart(b, b)

        def body(i, _, idx_v=idx_v, out=out, gstart=gstart):
            for b in range(NB):
                j = i * NB + b

                @pl.when(j < nch)
                def _(j=j, b=b):
                    pltpu.make_async_copy(
                        hc_hbm.at[idx_v.at[pl.ds(j * CH, CH)]], bufs[b],
                        sems[b]).wait()
                    pltpu.sync_copy(bufs[b],
                                    out.at[pl.ds(base + j * CH, CH)])

                @pl.when(j + NB < nch)
                def _(j=j, b=b):
                    gstart(j + NB, b)
            return 0
        lax.fori_loop(0, ngrp, body, 0)


@functools.cache
def _sc_gather_kernel():
    mesh = plsc.VectorSubcoreMesh(core_axis_name="c", subcore_axis_name="s")
    return pl.kernel(
        _sc_gather_body,
        mesh=mesh,
        out_type=[jax.ShapeDtypeStruct((TOT, H), jnp.float32)] * 2,
        scratch_types=(
            [pltpu.VMEM((MAXC * CH,), jnp.int32)] * 2
            + [pltpu.VMEM((CH, H), jnp.float32)] * NB
            + [pltpu.SemaphoreType.DMA] * NB
        ),
    )


BN = 2000
GRID = NN // BN


def _unpack(ref):
    w = lax.bitcast_convert_type(ref[...], jnp.uint32)
    hv = lax.bitcast_convert_type(w & jnp.uint32(0xFFFF0000), jnp.float32)
    cv = lax.bitcast_convert_type(w << 16, jnp.float32)
    return hv, cv


def _norm_body(iou_ref, n2_ref):
    x = iou_ref[...]
    n2 = jnp.sum(x * x, axis=1, keepdims=True)
    n2_ref[...] = n2.reshape(1, 1, BN)


def _norm_call(iou):
    return pl.pallas_call(
        _norm_body,
        grid=(GRID,),
        in_specs=[pl.BlockSpec((BN, 3 * H), lambda i: (i, 0))],
        out_specs=pl.BlockSpec((1, 1, BN), lambda i: (i, 0, 0)),
        out_shape=jax.ShapeDtypeStruct((GRID, 1, BN), jnp.float32),
    )(iou)


def _tc_body(g0_ref, g1_ref, n2_ref, w_ref, ufb_ref,
             biou_ref, siou_ref, sc_ref, ho_ref, co_ref):
    h0, c0 = _unpack(g0_ref)
    h1, c1 = _unpack(g1_ref)
    w = w_ref[...]
    h0b = h0.astype(jnp.bfloat16)
    h1b = h1.astype(jnp.bfloat16)
    y = (jnp.dot(h0b, w[:H, :], preferred_element_type=jnp.float32)
         + jnp.dot(h1b, w[H:, :], preferred_element_type=jnp.float32))
    ufb = ufb_ref[...]
    f0 = jax.nn.sigmoid(y[:, :H] + ufb[:, :H])
    f1 = jax.nn.sigmoid(y[:, H:2 * H] + ufb[:, H:])
    c_red = f0 * c0 + f1 * c1
    hnorm = jnp.sqrt(jnp.sum(h0 * h0 + h1 * h1, axis=1, keepdims=True))
    iounorm = jnp.sqrt(jnp.reshape(n2_ref[...], (BN, 1)))
    s = iounorm * siou_ref[0, 0] / jnp.maximum(hnorm, 1e-12)
    iou_b = y[:, 2 * H:] * s + biou_ref[...]
    c0norm = jnp.sqrt(jnp.sum(c0 * c0, axis=1, keepdims=True))
    crnorm = jnp.maximum(
        jnp.sqrt(jnp.sum(c_red * c_red, axis=1, keepdims=True)), 1e-12)
    c_data = c_red * (c0norm * sc_ref[0, 0] / crnorm)
    gi = jax.nn.sigmoid(iou_b[:, :H])
    go = jax.nn.sigmoid(iou_b[:, H:2 * H])
    gu = jnp.tanh(iou_b[:, 2 * H:])
    c_out = gi * gu + c_data
    co_ref[...] = c_out
    ho_ref[...] = go * jnp.tanh(c_out)


def _tc_call(g0, g1, n2, wcat, ufb, biou, siou, sc):
    row = lambda i: (i, 0)
    zero = lambda i: (0, 0)
    return pl.pallas_call(
        _tc_body,
        grid=(GRID,),
        in_specs=[
            pl.BlockSpec((BN, H), row),
            pl.BlockSpec((BN, H), row),
            pl.BlockSpec((1, 1, BN), lambda i: (i, 0, 0)),
            pl.BlockSpec((2 * H, 5 * H), zero),
            pl.BlockSpec((1, 2 * H), zero),
            pl.BlockSpec((1, 3 * H), zero),
            pl.BlockSpec((1, 1), zero, memory_space=pltpu.SMEM),
            pl.BlockSpec((1, 1), zero, memory_space=pltpu.SMEM),
        ],
        out_specs=[pl.BlockSpec((BN, H), row), pl.BlockSpec((BN, H), row)],
        out_shape=[jax.ShapeDtypeStruct((NN, H), jnp.float32)] * 2,
    )(g0, g1, n2, wcat, ufb, biou, siou, sc)


def kernel(h, c, iou, children, U_iou_w, b_iou, U_f_w, U_f_b, scale_iou,
           scale_c):
    hbits = lax.bitcast_convert_type(h, jnp.uint32)
    cbits = lax.bitcast_convert_type(c, jnp.uint32)
    hr = (hbits + jnp.uint32(0x8000)) & jnp.uint32(0xFFFF0000)
    cr = (cbits + jnp.uint32(0x8000)) >> 16
    hc = lax.bitcast_convert_type(hr | cr, jnp.float32)
    ch = children.astype(jnp.int32)
    pad = jnp.zeros((IDXPAD - NN,), jnp.int32)
    i0 = jnp.concatenate([ch[:, 0], pad])
    i1 = jnp.concatenate([ch[:, 1], pad])
    g0, g1 = _sc_gather_kernel()(hc, i0, i1)
    n2 = _norm_call(iou)
    wcat = jnp.concatenate([U_f_w.T, U_iou_w.T],
                           axis=1).astype(jnp.bfloat16)
    ufb = U_f_b.reshape(1, 2 * H)
    biou = b_iou.reshape(1, 3 * H)
    siou = scale_iou.reshape(1, 1)
    sc = scale_c.reshape(1, 1)
    h_out, c_out = _tc_call(g0, g1, n2, wcat, ufb, biou, siou, sc)
    return h_out, c_out


# speedup vs baseline: 10.7018x; 1.0685x over previous
import functools

import jax
import jax.numpy as jnp
from jax import lax
from jax.experimental import pallas as pl
from jax.experimental.pallas import tpu as pltpu
from jax.experimental.pallas import tpu_sc as plsc

H = 128
NN = 100000

NW = 32
CH = 128
NCH0 = 45
NCH1 = 5
MAXC = max(NCH0, NCH1)
TOT = 16 * (NCH0 + NCH1) * CH
IDXPAD = TOT + MAXC * CH

NB = 4


def _sc_gather_body(hc_hbm, i0_hbm, i1_hbm, o0, o1,
                    i0_v, i1_v, b0, b1, b2, b3, s0, s1, s2, s3):
    bufs = (b0, b1, b2, b3)
    sems = (s0, s1, s2, s3)
    cc = lax.axis_index("c")
    ss = lax.axis_index("s")
    nch = jnp.where(cc == 0, NCH0, NCH1)
    base = (jnp.where(cc == 0, ss * NCH0, 16 * NCH0 + ss * NCH1) * CH)
    ngrp = (nch + NB - 1) // NB
    pltpu.sync_copy(i0_hbm.at[pl.ds(base, MAXC * CH)], i0_v)
    pltpu.sync_copy(i1_hbm.at[pl.ds(base, MAXC * CH)], i1_v)
    for idx_v, out in ((i0_v, o0), (i1_v, o1)):
        def gstart(j, b, idx_v=idx_v):
            pltpu.async_copy(
                hc_hbm.at[idx_v.at[pl.ds(j * CH, CH)]], bufs[b], sems[b])

        for b in range(NB):
            @pl.when(b < nch)
            def _(b=b, gstart=gstart):
                gstart(b, b)

        def body(i, _, idx_v=idx_v, out=out, gstart=gstart):
            for b in range(NB):
                j = i * NB + b

                @pl.when(j < nch)
                def _(j=j, b=b):
                    pltpu.make_async_copy(
                        hc_hbm.at[idx_v.at[pl.ds(j * CH, CH)]], bufs[b],
                        sems[b]).wait()
                    pltpu.sync_copy(bufs[b],
                                    out.at[pl.ds(base + j * CH, CH)])

                @pl.when(j + NB < nch)
                def _(j=j, b=b):
                    gstart(j + NB, b)
            return 0
        lax.fori_loop(0, ngrp, body, 0)


@functools.cache
def _sc_gather_kernel():
    mesh = plsc.VectorSubcoreMesh(core_axis_name="c", subcore_axis_name="s")
    return pl.kernel(
        _sc_gather_body,
        mesh=mesh,
        out_type=[jax.ShapeDtypeStruct((TOT, H), jnp.float32)] * 2,
        scratch_types=(
            [pltpu.VMEM((MAXC * CH,), jnp.int32)] * 2
            + [pltpu.VMEM((CH, H), jnp.float32)] * NB
            + [pltpu.SemaphoreType.DMA] * NB
        ),
    )


BN = 2000
GRID = NN // BN


def _unpack(ref):
    w = lax.bitcast_convert_type(ref[...], jnp.uint32)
    hv = lax.bitcast_convert_type(w & jnp.uint32(0xFFFF0000), jnp.float32)
    cv = lax.bitcast_convert_type(w << 16, jnp.float32)
    return hv, cv


def _tc_body(g0_ref, g1_ref, iou_ref, w_ref, ufb_ref,
             biou_ref, siou_ref, sc_ref, ho_ref, co_ref):
    h0, c0 = _unpack(g0_ref)
    h1, c1 = _unpack(g1_ref)
    w = w_ref[...]
    h0b = h0.astype(jnp.bfloat16)
    h1b = h1.astype(jnp.bfloat16)
    y = (jnp.dot(h0b, w[:H, :], preferred_element_type=jnp.float32)
         + jnp.dot(h1b, w[H:, :], preferred_element_type=jnp.float32))
    ufb = ufb_ref[...]
    f0 = jax.nn.sigmoid(y[:, :H] + ufb[:, :H])
    f1 = jax.nn.sigmoid(y[:, H:2 * H] + ufb[:, H:])
    c_red = f0 * c0 + f1 * c1
    hnorm = jnp.sqrt(jnp.sum(h0 * h0 + h1 * h1, axis=1, keepdims=True))
    iou = iou_ref[...]
    iounorm = jnp.sqrt(jnp.sum(iou * iou, axis=1, keepdims=True))
    s = iounorm * siou_ref[0, 0] / jnp.maximum(hnorm, 1e-12)
    iou_b = y[:, 2 * H:] * s + biou_ref[...]
    c0norm = jnp.sqrt(jnp.sum(c0 * c0, axis=1, keepdims=True))
    crnorm = jnp.maximum(
        jnp.sqrt(jnp.sum(c_red * c_red, axis=1, keepdims=True)), 1e-12)
    c_data = c_red * (c0norm * sc_ref[0, 0] / crnorm)
    gi = jax.nn.sigmoid(iou_b[:, :H])
    go = jax.nn.sigmoid(iou_b[:, H:2 * H])
    gu = jnp.tanh(iou_b[:, 2 * H:])
    c_out = gi * gu + c_data
    co_ref[...] = c_out
    ho_ref[...] = go * jnp.tanh(c_out)


def _tc_call(g0, g1, iou, wcat, ufb, biou, siou, sc):
    row = lambda i: (i, 0)
    zero = lambda i: (0, 0)
    return pl.pallas_call(
        _tc_body,
        grid=(GRID,),
        in_specs=[
            pl.BlockSpec((BN, H), row),
            pl.BlockSpec((BN, H), row),
            pl.BlockSpec((BN, 3 * H), row),
            pl.BlockSpec((2 * H, 5 * H), zero),
            pl.BlockSpec((1, 2 * H), zero),
            pl.BlockSpec((1, 3 * H), zero),
            pl.BlockSpec((1, 1), zero, memory_space=pltpu.SMEM),
            pl.BlockSpec((1, 1), zero, memory_space=pltpu.SMEM),
        ],
        out_specs=[pl.BlockSpec((BN, H), row), pl.BlockSpec((BN, H), row)],
        out_shape=[jax.ShapeDtypeStruct((NN, H), jnp.float32)] * 2,
    )(g0, g1, iou, wcat, ufb, biou, siou, sc)


def kernel(h, c, iou, children, U_iou_w, b_iou, U_f_w, U_f_b, scale_iou,
           scale_c):
    hbits = lax.bitcast_convert_type(h, jnp.uint32)
    cbits = lax.bitcast_convert_type(c, jnp.uint32)
    hr = (hbits + jnp.uint32(0x8000)) & jnp.uint32(0xFFFF0000)
    cr = (cbits + jnp.uint32(0x8000)) >> 16
    hc = lax.bitcast_convert_type(hr | cr, jnp.float32)
    ch = children.astype(jnp.int32)
    pad = jnp.zeros((IDXPAD - NN,), jnp.int32)
    i0 = jnp.concatenate([ch[:, 0], pad])
    i1 = jnp.concatenate([ch[:, 1], pad])
    g0, g1 = _sc_gather_kernel()(hc, i0, i1)
    wcat = jnp.concatenate([U_f_w.T, U_iou_w.T],
                           axis=1).astype(jnp.bfloat16)
    ufb = U_f_b.reshape(1, 2 * H)
    biou = b_iou.reshape(1, 3 * H)
    siou = scale_iou.reshape(1, 1)
    sc = scale_c.reshape(1, 1)
    h_out, c_out = _tc_call(g0, g1, iou, wcat, ufb, biou, siou, sc)
    return h_out, c_out
